# baseline (speedup 1.0000x reference)
import jax
import jax.numpy as jnp
from jax.experimental import pallas as pl

ITEMS = 100000
D = 32
B = 8
PACK = 128 // D
ROWS = ITEMS // PACK
BLKR = 1000


def _blend_body(x_ref, e_ref, a_ref, o_ref):
    x = x_ref[...]
    e = e_ref[...]
    a = a_ref[...]
    o_ref[...] = (e + a * (x - e[None, :, :]))


def kernel(nodes_output, embedding_table, alpha):
    nodes = nodes_output.reshape(B, ROWS, PACK * D)
    emb = embedding_table.reshape(ROWS, PACK * D)
    gate = jnp.broadcast_to(
        alpha.reshape(ROWS, PACK, 1), (ROWS, PACK, D)
    ).reshape(ROWS, PACK * D)
    out = pl.pallas_call(
        _blend_body,
        grid=(ROWS // BLKR,),
        in_specs=[
            pl.BlockSpec((B, BLKR, PACK * D), lambda i: (0, i, 0)),
            pl.BlockSpec((BLKR, PACK * D), lambda i: (i, 0)),
            pl.BlockSpec((BLKR, PACK * D), lambda i: (i, 0)),
        ],
        out_specs=pl.BlockSpec((B, BLKR, PACK * D), lambda i: (0, i, 0)),
        out_shape=jax.ShapeDtypeStruct((B, ROWS, PACK * D), jnp.float32),
    )(nodes, emb, gate)
    return out.reshape(B, ITEMS, D)
